# Initial kernel scaffold; baseline (speedup 1.0000x reference)
import functools

import jax
import jax.numpy as jnp
from jax import lax
from jax.experimental import pallas as pl
from jax.experimental.pallas import tpu as pltpu
from jax.experimental.pallas import tpu_sc as plsc

N_NODES = 10000
D = 128

NC = 2
NS = 16
NWK = NC * NS

CHUNK = 128
N_PAD = 10240
DAUG = 144
BS = 512


def _make_sc_scatter(width: int, e_pad: int):
    npw = e_pad // NWK
    nch = npw // CHUNK
    assert npw % CHUNK == 0 and nch % 2 == 0
    rpt = N_PAD // NS

    mesh = plsc.VectorSubcoreMesh(core_axis_name="c", subcore_axis_name="s")

    @functools.partial(
        pl.kernel,
        out_type=jax.ShapeDtypeStruct((NC, N_PAD, width), jnp.float32),
        mesh=mesh,
        scratch_types=[
            pltpu.VMEM((2, CHUNK), jnp.int32),
            pltpu.VMEM((2, CHUNK), jnp.int32),
            pltpu.VMEM((2, CHUNK, width), jnp.float32),
            pltpu.VMEM_SHARED((N_PAD, width), jnp.float32),
            pltpu.SemaphoreType.DMA,
            pltpu.SemaphoreType.DMA,
        ],
    )
    def sc_kernel(tab, colh, rowh, zer, out, colv, rowv, rowsv, acc, s0, s1):
        core = lax.axis_index("c")
        sub = lax.axis_index("s")
        wid = core * NS + sub
        base = wid * npw
        r0 = sub * rpt

        pltpu.sync_copy(zer.at[pl.ds(r0, rpt)], acc.at[pl.ds(r0, rpt)])
        plsc.subcore_barrier()

        sems = (s0, s1)

        def load_idx(j, b):
            pltpu.sync_copy(colh.at[pl.ds(base + j * CHUNK, CHUNK)], colv.at[b])
            pltpu.sync_copy(rowh.at[pl.ds(base + j * CHUNK, CHUNK)], rowv.at[b])

        def start_gather(b):
            pltpu.async_copy(tab.at[colv.at[b]], rowsv.at[b], sems[b])

        def wait_gather(b):
            pltpu.make_async_copy(tab.at[colv.at[b]], rowsv.at[b], sems[b]).wait()

        def scatter_add(b):
            pltpu.sync_copy(rowsv.at[b], acc.at[rowv.at[b]], add=True)

        load_idx(0, 0)
        start_gather(0)

        def body(i, _):
            g = 2 * i
            load_idx(g + 1, 1)
            start_gather(1)
            wait_gather(0)
            scatter_add(0)

            @pl.when(g + 2 < nch)
            def _prefetch():
                load_idx(g + 2, 0)
                start_gather(0)

            wait_gather(1)
            scatter_add(1)
            return ()

        lax.fori_loop(0, nch // 2, body, ())

        plsc.subcore_barrier()
        pltpu.sync_copy(acc.at[pl.ds(r0, rpt)], out.at[core].at[pl.ds(r0, rpt)])

    return sc_kernel


def _layer1_tc(p, w1t, b1):
    grid = N_PAD // BS

    def body(p_ref, w_ref, b_ref, h_ref, rd_ref):
        s = p_ref[0] + p_ref[1]
        agg = s[:, :D]
        cnt = s[:, D:D + 1]
        rd = 1.0 / jnp.maximum(cnt, 1.0)
        rdb = jnp.broadcast_to(rd, (BS, D))
        h = jnp.dot(agg * rdb, w_ref[...], preferred_element_type=jnp.float32)
        h_ref[...] = jnp.maximum(h + b_ref[...], 0.0)
        rd_ref[...] = rdb

    return pl.pallas_call(
        body,
        grid=(grid,),
        in_specs=[
            pl.BlockSpec((NC, BS, DAUG), lambda i: (0, i, 0)),
            pl.BlockSpec((D, D), lambda i: (0, 0)),
            pl.BlockSpec((1, D), lambda i: (0, 0)),
        ],
        out_specs=[
            pl.BlockSpec((BS, D), lambda i: (i, 0)),
            pl.BlockSpec((BS, D), lambda i: (i, 0)),
        ],
        out_shape=[
            jax.ShapeDtypeStruct((N_PAD, D), jnp.float32),
            jax.ShapeDtypeStruct((N_PAD, D), jnp.float32),
        ],
    )(p, w1t, b1)


def _layer2_tc(p, rdeg, w2t, b2):
    grid = N_PAD // BS

    def body(p_ref, rd_ref, w_ref, b_ref, o_ref):
        s = (p_ref[0] + p_ref[1]) * rd_ref[...]
        o = jnp.dot(s, w_ref[...], preferred_element_type=jnp.float32)
        o_ref[...] = jnp.maximum(o + b_ref[...], 0.0)

    return pl.pallas_call(
        body,
        grid=(grid,),
        in_specs=[
            pl.BlockSpec((NC, BS, D), lambda i: (0, i, 0)),
            pl.BlockSpec((BS, D), lambda i: (i, 0)),
            pl.BlockSpec((D, D), lambda i: (0, 0)),
            pl.BlockSpec((1, D), lambda i: (0, 0)),
        ],
        out_specs=pl.BlockSpec((BS, D), lambda i: (i, 0)),
        out_shape=jax.ShapeDtypeStruct((N_PAD, D), jnp.float32),
    )(p, rdeg, w2t, b2)


@jax.jit
def kernel(x, edge_index, W1, b1, W2, b2):
    n = x.shape[0]
    e = edge_index.shape[1]

    row = edge_index[0].astype(jnp.int32)
    col = edge_index[1].astype(jnp.int32)

    e_pad = -(-e // (NWK * 2 * CHUNK)) * (NWK * 2 * CHUNK)
    row_p = jnp.concatenate(
        [row, jnp.full((e_pad - e,), N_NODES, dtype=jnp.int32)])
    col_p = jnp.concatenate([col, jnp.zeros((e_pad - e,), dtype=jnp.int32)])

    xa = jnp.zeros((N_PAD, DAUG), dtype=jnp.float32)
    xa = xa.at[:n, :D].set(x)
    xa = xa.at[:n, D].set(1.0)

    z_aug = jnp.zeros((N_PAD, DAUG), dtype=jnp.float32)
    z_d = jnp.zeros((N_PAD, D), dtype=jnp.float32)

    sc1 = _make_sc_scatter(DAUG, e_pad)
    sc2 = _make_sc_scatter(D, e_pad)

    p1 = sc1(xa, col_p, row_p, z_aug)
    h, rdeg = _layer1_tc(p1, W1.T, b1.reshape(1, D))

    p2 = sc2(h, col_p, row_p, z_d)
    out = _layer2_tc(p2, rdeg, W2.T, b2.reshape(1, D))

    return out[:n]


# speedup vs baseline: 3.1093x; 3.1093x over previous
import functools

import jax
import jax.numpy as jnp
from jax import lax
from jax.experimental import pallas as pl
from jax.experimental.pallas import tpu as pltpu
from jax.experimental.pallas import tpu_sc as plsc

N_NODES = 10000
D = 128

NC = 2
NS = 16
NWK = NC * NS

CHUNK = 128
N_PAD = 10240
DAUG = 144
BS = 512


def _make_sc_scatter(width: int, e_pad: int):
    npw = e_pad // NWK
    nch = npw // CHUNK
    assert npw % CHUNK == 0 and nch % 2 == 0
    rpt = N_PAD // NS

    mesh = plsc.VectorSubcoreMesh(
        core_axis_name="c", subcore_axis_name="s", num_cores=NC, num_subcores=NS)

    @functools.partial(
        pl.kernel,
        out_type=jax.ShapeDtypeStruct((NC, N_PAD, width), jnp.float32),
        mesh=mesh,
        scratch_types=[
            pltpu.VMEM((2, CHUNK), jnp.int32),
            pltpu.VMEM((2, CHUNK), jnp.int32),
            pltpu.VMEM((2, CHUNK, width), jnp.float32),
            pltpu.VMEM_SHARED((N_PAD, width), jnp.float32),
            pltpu.SemaphoreType.DMA,
            pltpu.SemaphoreType.DMA,
        ],
        compiler_params=pltpu.CompilerParams(use_tc_tiling_on_sc=False),
    )
    def sc_kernel(tab, colh, rowh, zer, out, colv, rowv, rowsv, acc, s0, s1):
        core = lax.axis_index("c")
        sub = lax.axis_index("s")
        wid = core * NS + sub
        base = wid * npw
        r0 = sub * rpt

        pltpu.sync_copy(zer.at[pl.ds(r0, rpt)], acc.at[pl.ds(r0, rpt)])
        plsc.subcore_barrier()

        sems = (s0, s1)

        def load_idx(j, b):
            pltpu.sync_copy(colh.at[pl.ds(base + j * CHUNK, CHUNK)], colv.at[b])
            pltpu.sync_copy(rowh.at[pl.ds(base + j * CHUNK, CHUNK)], rowv.at[b])

        def start_gather(b):
            pltpu.async_copy(tab.at[colv.at[b]], rowsv.at[b], sems[b])

        def wait_gather(b):
            pltpu.make_async_copy(tab.at[colv.at[b]], rowsv.at[b], sems[b]).wait()

        def scatter_add(b):
            pltpu.sync_copy(rowsv.at[b], acc.at[rowv.at[b]], add=True)

        load_idx(0, 0)
        start_gather(0)

        def body(i, _):
            g = 2 * i
            load_idx(g + 1, 1)
            start_gather(1)
            wait_gather(0)
            scatter_add(0)

            @pl.when(g + 2 < nch)
            def _prefetch():
                load_idx(g + 2, 0)
                start_gather(0)

            wait_gather(1)
            scatter_add(1)
            return ()

        lax.fori_loop(0, nch // 2, body, ())

        plsc.subcore_barrier()
        pltpu.sync_copy(acc.at[pl.ds(r0, rpt)], out.at[core].at[pl.ds(r0, rpt)])

    return sc_kernel


def _layer1_tc(p, w1t, b1):
    grid = N_PAD // BS

    def body(p_ref, w_ref, b_ref, h_ref, rd_ref):
        s = p_ref[0] + p_ref[1]
        agg = s[:, :D]
        cnt = s[:, D:D + 1]
        rd = 1.0 / jnp.maximum(cnt, 1.0)
        rdb = jnp.broadcast_to(rd, (BS, D))
        h = jnp.dot(agg * rdb, w_ref[...], preferred_element_type=jnp.float32)
        h_ref[...] = jnp.maximum(h + b_ref[...], 0.0)
        rd_ref[...] = rdb

    return pl.pallas_call(
        body,
        grid=(grid,),
        in_specs=[
            pl.BlockSpec((NC, BS, DAUG), lambda i: (0, i, 0)),
            pl.BlockSpec((D, D), lambda i: (0, 0)),
            pl.BlockSpec((1, D), lambda i: (0, 0)),
        ],
        out_specs=[
            pl.BlockSpec((BS, D), lambda i: (i, 0)),
            pl.BlockSpec((BS, D), lambda i: (i, 0)),
        ],
        out_shape=[
            jax.ShapeDtypeStruct((N_PAD, D), jnp.float32),
            jax.ShapeDtypeStruct((N_PAD, D), jnp.float32),
        ],
    )(p, w1t, b1)


def _layer2_tc(p, rdeg, w2t, b2):
    grid = N_PAD // BS

    def body(p_ref, rd_ref, w_ref, b_ref, o_ref):
        s = (p_ref[0] + p_ref[1]) * rd_ref[...]
        o = jnp.dot(s, w_ref[...], preferred_element_type=jnp.float32)
        o_ref[...] = jnp.maximum(o + b_ref[...], 0.0)

    return pl.pallas_call(
        body,
        grid=(grid,),
        in_specs=[
            pl.BlockSpec((NC, BS, D), lambda i: (0, i, 0)),
            pl.BlockSpec((BS, D), lambda i: (i, 0)),
            pl.BlockSpec((D, D), lambda i: (0, 0)),
            pl.BlockSpec((1, D), lambda i: (0, 0)),
        ],
        out_specs=pl.BlockSpec((BS, D), lambda i: (i, 0)),
        out_shape=jax.ShapeDtypeStruct((N_PAD, D), jnp.float32),
    )(p, rdeg, w2t, b2)


@jax.jit
def kernel(x, edge_index, W1, b1, W2, b2):
    n = x.shape[0]
    e = edge_index.shape[1]

    row = edge_index[0].astype(jnp.int32)
    col = edge_index[1].astype(jnp.int32)

    e_pad = -(-e // (NWK * 2 * CHUNK)) * (NWK * 2 * CHUNK)
    row_p = jnp.concatenate(
        [row, jnp.full((e_pad - e,), N_NODES, dtype=jnp.int32)])
    col_p = jnp.concatenate([col, jnp.zeros((e_pad - e,), dtype=jnp.int32)])

    xa = jnp.zeros((N_PAD, DAUG), dtype=jnp.float32)
    xa = xa.at[:n, :D].set(x)
    xa = xa.at[:n, D].set(1.0)

    z_aug = jnp.zeros((N_PAD, DAUG), dtype=jnp.float32)
    z_d = jnp.zeros((N_PAD, D), dtype=jnp.float32)

    sc1 = _make_sc_scatter(DAUG, e_pad)
    sc2 = _make_sc_scatter(D, e_pad)

    p1 = sc1(xa, col_p, row_p, z_aug)
    h, rdeg = _layer1_tc(p1, W1.T, b1.reshape(1, D))

    p2 = sc2(h, col_p, row_p, z_d)
    out = _layer2_tc(p2, rdeg, W2.T, b2.reshape(1, D))

    return out[:n]


# speedup vs baseline: 7.0924x; 2.2810x over previous
import functools

import jax
import jax.numpy as jnp
from jax import lax
from jax.experimental import pallas as pl
from jax.experimental.pallas import tpu as pltpu
from jax.experimental.pallas import tpu_sc as plsc

N_NODES = 10000
D = 128

NC = 2
NS = 16
NWK = NC * NS

CHUNK = 128
N_PAD = 10240
DAUG = 144
BS = 512


def _make_sc_scatter(width: int, e_pad: int):
    npw = e_pad // NWK
    nch = npw // CHUNK
    assert npw % CHUNK == 0 and nch % 2 == 0
    rpt = N_PAD // NS

    mesh = plsc.VectorSubcoreMesh(
        core_axis_name="c", subcore_axis_name="s", num_cores=NC, num_subcores=NS)

    @functools.partial(
        pl.kernel,
        out_type=jax.ShapeDtypeStruct((NC, N_PAD, width), jnp.float32),
        mesh=mesh,
        scratch_types=[
            pltpu.VMEM((2, CHUNK), jnp.int32),
            pltpu.VMEM((2, CHUNK), jnp.int32),
            pltpu.VMEM((2, CHUNK, width), jnp.float32),
            pltpu.VMEM_SHARED((N_PAD, width), jnp.float32),
            pltpu.SemaphoreType.DMA,
            pltpu.SemaphoreType.DMA,
        ],
        compiler_params=pltpu.CompilerParams(use_tc_tiling_on_sc=False),
    )
    def sc_kernel(tab, colh, rowh, zer, out, colv, rowv, rowsv, acc, s0, s1):
        core = lax.axis_index("c")
        sub = lax.axis_index("s")
        wid = core * NS + sub
        base = wid * npw
        r0 = sub * rpt

        pltpu.sync_copy(zer.at[pl.ds(r0, rpt)], acc.at[pl.ds(r0, rpt)])
        plsc.subcore_barrier()

        sems = (s0, s1)

        def load_idx(j, b):
            pltpu.sync_copy(colh.at[pl.ds(base + j * CHUNK, CHUNK)], colv.at[b])
            pltpu.sync_copy(rowh.at[pl.ds(base + j * CHUNK, CHUNK)], rowv.at[b])

        def start_gather(b):
            pltpu.async_copy(tab.at[colv.at[b]], rowsv.at[b], sems[b])

        def wait_gather(b):
            pltpu.make_async_copy(tab.at[colv.at[b]], rowsv.at[b], sems[b]).wait()

        def scatter_add(b):
            pltpu.sync_copy(rowsv.at[b], acc.at[rowv.at[b]], add=True)

        load_idx(0, 0)
        start_gather(0)

        def body(i, _):
            g = 2 * i
            load_idx(g + 1, 1)
            start_gather(1)
            wait_gather(0)
            scatter_add(0)

            @pl.when(g + 2 < nch)
            def _prefetch():
                load_idx(g + 2, 0)
                start_gather(0)

            wait_gather(1)
            scatter_add(1)
            return ()

        lax.fori_loop(0, nch // 2, body, ())

        plsc.subcore_barrier()
        pltpu.sync_copy(acc.at[pl.ds(r0, rpt)], out.at[core].at[pl.ds(r0, rpt)])

    return sc_kernel


def _layer1_tc(p, w1t, b1):
    grid = N_PAD // BS

    def body(p_ref, w_ref, b_ref, h_ref, rd_ref):
        s = p_ref[0] + p_ref[1]
        agg = s[:, :D]
        cnt = s[:, D:D + 1]
        rd = 1.0 / jnp.maximum(cnt, 1.0)
        rdb = jnp.broadcast_to(rd, (BS, D))
        h = jnp.dot(agg * rdb, w_ref[...], preferred_element_type=jnp.float32)
        h_ref[...] = jnp.maximum(h + b_ref[...], 0.0)
        rd_ref[...] = rdb

    return pl.pallas_call(
        body,
        grid=(grid,),
        in_specs=[
            pl.BlockSpec((NC, BS, DAUG), lambda i: (0, i, 0)),
            pl.BlockSpec((D, D), lambda i: (0, 0)),
            pl.BlockSpec((1, D), lambda i: (0, 0)),
        ],
        out_specs=[
            pl.BlockSpec((BS, D), lambda i: (i, 0)),
            pl.BlockSpec((BS, D), lambda i: (i, 0)),
        ],
        out_shape=[
            jax.ShapeDtypeStruct((N_PAD, D), jnp.float32),
            jax.ShapeDtypeStruct((N_PAD, D), jnp.float32),
        ],
    )(p, w1t, b1)


def _layer2_tc(p, rdeg, w2t, b2):
    grid = N_PAD // BS

    def body(p_ref, rd_ref, w_ref, b_ref, o_ref):
        s = (p_ref[0] + p_ref[1]) * rd_ref[...]
        o = jnp.dot(s, w_ref[...], preferred_element_type=jnp.float32)
        o_ref[...] = jnp.maximum(o + b_ref[...], 0.0)

    return pl.pallas_call(
        body,
        grid=(grid,),
        in_specs=[
            pl.BlockSpec((NC, BS, D), lambda i: (0, i, 0)),
            pl.BlockSpec((BS, D), lambda i: (i, 0)),
            pl.BlockSpec((D, D), lambda i: (0, 0)),
            pl.BlockSpec((1, D), lambda i: (0, 0)),
        ],
        out_specs=pl.BlockSpec((BS, D), lambda i: (i, 0)),
        out_shape=jax.ShapeDtypeStruct((N_PAD, D), jnp.float32),
    )(p, rdeg, w2t, b2)


@jax.jit
def kernel(x, edge_index, W1, b1, W2, b2):
    n = x.shape[0]
    e = edge_index.shape[1]

    row = edge_index[0].astype(jnp.int32)
    col = edge_index[1].astype(jnp.int32)

    e_pad = -(-e // (NWK * 2 * CHUNK)) * (NWK * 2 * CHUNK)
    npad = e_pad - e
    pad_rows = N_NODES + (jnp.arange(npad, dtype=jnp.int32) % (N_PAD - N_NODES))
    pad_cols = jnp.arange(npad, dtype=jnp.int32) % n
    row_p = jnp.concatenate([row, pad_rows])
    col_p = jnp.concatenate([col, pad_cols])

    xa = jnp.zeros((N_PAD, DAUG), dtype=jnp.float32)
    xa = xa.at[:n, :D].set(x)
    xa = xa.at[:n, D].set(1.0)

    z_aug = jnp.zeros((N_PAD, DAUG), dtype=jnp.float32)
    z_d = jnp.zeros((N_PAD, D), dtype=jnp.float32)

    sc1 = _make_sc_scatter(DAUG, e_pad)
    sc2 = _make_sc_scatter(D, e_pad)

    p1 = sc1(xa, col_p, row_p, z_aug)
    h, rdeg = _layer1_tc(p1, W1.T, b1.reshape(1, D))

    p2 = sc2(h, col_p, row_p, z_d)
    out = _layer2_tc(p2, rdeg, W2.T, b2.reshape(1, D))

    return out[:n]


# speedup vs baseline: 8.0167x; 1.1303x over previous
import functools

import jax
import jax.numpy as jnp
from jax import lax
from jax.experimental import pallas as pl
from jax.experimental.pallas import tpu as pltpu
from jax.experimental.pallas import tpu_sc as plsc

N_NODES = 10000
D = 128

NC = 2
NS = 16
NWK = NC * NS

CHUNK = 128
GRP = 4
N_PAD = 10112
DAUG = 144
BS = 632


def _make_sc_scatter(width: int, e_pad: int):
    npw = e_pad // NWK
    nch = npw // CHUNK
    assert npw % CHUNK == 0 and nch % 2 == 0
    rpt = N_PAD // NS

    mesh = plsc.VectorSubcoreMesh(
        core_axis_name="c", subcore_axis_name="s", num_cores=NC, num_subcores=NS)

    @functools.partial(
        pl.kernel,
        out_type=jax.ShapeDtypeStruct((NC, N_PAD, width), jnp.float32),
        mesh=mesh,
        scratch_types=[
            pltpu.VMEM((2, GRP, CHUNK), jnp.int32),
            pltpu.VMEM((2, GRP, CHUNK), jnp.int32),
            pltpu.VMEM((2, CHUNK, width), jnp.float32),
            pltpu.VMEM_SHARED((N_PAD, width), jnp.float32),
            pltpu.SemaphoreType.DMA,
            pltpu.SemaphoreType.DMA,
        ],
        compiler_params=pltpu.CompilerParams(use_tc_tiling_on_sc=False),
    )
    def sc_kernel(tab, colh, rowh, zer, out, colv, rowv, rowsv, acc, s0, s1):
        core = lax.axis_index("c")
        sub = lax.axis_index("s")
        wid = core * NS + sub
        r0 = sub * rpt

        pltpu.sync_copy(zer.at[pl.ds(r0, rpt)], acc.at[pl.ds(r0, rpt)])
        plsc.subcore_barrier()

        sems = (s0, s1)

        def load_group(j):
            q = j // GRP
            s = q % 2
            pltpu.sync_copy(colh.at[pl.ds(wid * nch + q * GRP, GRP)], colv.at[s])
            pltpu.sync_copy(rowh.at[pl.ds(wid * nch + q * GRP, GRP)], rowv.at[s])

        def start_gather(j, b):
            pltpu.async_copy(
                tab.at[colv.at[(j // GRP) % 2, j % GRP]], rowsv.at[b], sems[b])

        def wait_gather(b):
            pltpu.make_async_copy(tab.at[colv.at[0, 0]], rowsv.at[b],
                                  sems[b]).wait()

        def scatter_add(j, b):
            pltpu.sync_copy(rowsv.at[b],
                            acc.at[rowv.at[(j // GRP) % 2, j % GRP]], add=True)

        load_group(0)
        start_gather(0, 0)

        def body(i, _):
            g = 2 * i
            start_gather(g + 1, 1)
            wait_gather(0)
            scatter_add(g, 0)

            @pl.when(jnp.logical_and(g % GRP == 2, g + 2 < nch))
            def _load_next():
                load_group(g + 2)

            @pl.when(g + 2 < nch)
            def _prefetch():
                start_gather(g + 2, 0)

            wait_gather(1)
            scatter_add(g + 1, 1)
            return ()

        lax.fori_loop(0, nch // 2, body, ())

        plsc.subcore_barrier()
        pltpu.sync_copy(acc.at[pl.ds(r0, rpt)], out.at[core].at[pl.ds(r0, rpt)])

    return sc_kernel


def _layer1_tc(p, w1t, b1):
    grid = N_PAD // BS

    def body(p_ref, w_ref, b_ref, h_ref, rd_ref):
        s = p_ref[0] + p_ref[1]
        agg = s[:, :D]
        cnt = s[:, D:D + 1]
        rd = 1.0 / jnp.maximum(cnt, 1.0)
        rdb = jnp.broadcast_to(rd, (BS, D))
        h = jnp.dot(agg * rdb, w_ref[...], preferred_element_type=jnp.float32)
        h_ref[...] = jnp.maximum(h + b_ref[...], 0.0)
        rd_ref[...] = rdb

    return pl.pallas_call(
        body,
        grid=(grid,),
        in_specs=[
            pl.BlockSpec((NC, BS, DAUG), lambda i: (0, i, 0)),
            pl.BlockSpec((D, D), lambda i: (0, 0)),
            pl.BlockSpec((1, D), lambda i: (0, 0)),
        ],
        out_specs=[
            pl.BlockSpec((BS, D), lambda i: (i, 0)),
            pl.BlockSpec((BS, D), lambda i: (i, 0)),
        ],
        out_shape=[
            jax.ShapeDtypeStruct((N_PAD, D), jnp.float32),
            jax.ShapeDtypeStruct((N_PAD, D), jnp.float32),
        ],
    )(p, w1t, b1)


def _layer2_tc(p, rdeg, w2t, b2):
    grid = N_PAD // BS

    def body(p_ref, rd_ref, w_ref, b_ref, o_ref):
        s = (p_ref[0] + p_ref[1]) * rd_ref[...]
        o = jnp.dot(s, w_ref[...], preferred_element_type=jnp.float32)
        o_ref[...] = jnp.maximum(o + b_ref[...], 0.0)

    return pl.pallas_call(
        body,
        grid=(grid,),
        in_specs=[
            pl.BlockSpec((NC, BS, D), lambda i: (0, i, 0)),
            pl.BlockSpec((BS, D), lambda i: (i, 0)),
            pl.BlockSpec((D, D), lambda i: (0, 0)),
            pl.BlockSpec((1, D), lambda i: (0, 0)),
        ],
        out_specs=pl.BlockSpec((BS, D), lambda i: (i, 0)),
        out_shape=jax.ShapeDtypeStruct((N_PAD, D), jnp.float32),
    )(p, rdeg, w2t, b2)


@jax.jit
def kernel(x, edge_index, W1, b1, W2, b2):
    n = x.shape[0]
    e = edge_index.shape[1]

    row = edge_index[0].astype(jnp.int32)
    col = edge_index[1].astype(jnp.int32)

    e_pad = -(-e // (NWK * 2 * CHUNK)) * (NWK * 2 * CHUNK)
    npad = e_pad - e
    pad_rows = N_NODES + (jnp.arange(npad, dtype=jnp.int32) % (N_PAD - N_NODES))
    pad_cols = jnp.arange(npad, dtype=jnp.int32) % n
    row_p = jnp.concatenate([row, pad_rows]).reshape(e_pad // CHUNK, CHUNK)
    col_p = jnp.concatenate([col, pad_cols]).reshape(e_pad // CHUNK, CHUNK)

    xa = jnp.zeros((N_PAD, DAUG), dtype=jnp.float32)
    xa = xa.at[:n, :D].set(x)
    xa = xa.at[:n, D].set(1.0)

    z_aug = jnp.zeros((N_PAD, DAUG), dtype=jnp.float32)
    z_d = jnp.zeros((N_PAD, D), dtype=jnp.float32)

    sc1 = _make_sc_scatter(DAUG, e_pad)
    sc2 = _make_sc_scatter(D, e_pad)

    p1 = sc1(xa, col_p, row_p, z_aug)
    h, rdeg = _layer1_tc(p1, W1.T, b1.reshape(1, D))

    p2 = sc2(h, col_p, row_p, z_d)
    out = _layer2_tc(p2, rdeg, W2.T, b2.reshape(1, D))

    return out[:n]


# speedup vs baseline: 8.7984x; 1.0975x over previous
import functools

import jax
import jax.numpy as jnp
from jax import lax
from jax.experimental import pallas as pl
from jax.experimental.pallas import tpu as pltpu
from jax.experimental.pallas import tpu_sc as plsc

N_NODES = 10000
D = 128

NC = 2
NS = 16
NWK = NC * NS

CHUNK = 128
GRP = 4
N_PAD = 10112
BS2 = 632


def _make_sc_scatter(e_pad: int, with_deg: bool):
    npw = e_pad // NWK
    nch = npw // CHUNK
    assert npw % CHUNK == 0 and nch % GRP == 0 and nch % 2 == 0
    rpt = N_PAD // NS

    mesh = plsc.VectorSubcoreMesh(
        core_axis_name="c", subcore_axis_name="s", num_cores=NC, num_subcores=NS)

    out_type = [jax.ShapeDtypeStruct((NC, N_PAD, D), jnp.float32)]
    scratch = [
        pltpu.VMEM((2, GRP, CHUNK), jnp.int32),
        pltpu.VMEM((2, GRP, CHUNK), jnp.int32),
        pltpu.VMEM((2, CHUNK, D), jnp.float32),
        pltpu.VMEM_SHARED((N_PAD, D), jnp.float32),
        pltpu.SemaphoreType.DMA,
        pltpu.SemaphoreType.DMA,
    ]
    if with_deg:
        out_type.append(jax.ShapeDtypeStruct((NC, N_PAD), jnp.float32))
        scratch += [
            pltpu.VMEM_SHARED((N_PAD,), jnp.float32),
            pltpu.VMEM((CHUNK,), jnp.float32),
        ]

    @functools.partial(
        pl.kernel,
        out_type=out_type,
        mesh=mesh,
        scratch_types=scratch,
        compiler_params=pltpu.CompilerParams(use_tc_tiling_on_sc=False),
    )
    def sc_kernel(tab, colh, rowh, zer, zer1, *refs):
        if with_deg:
            out, outd, colv, rowv, rowsv, acc, s0, s1, accd, ones = refs
        else:
            out, colv, rowv, rowsv, acc, s0, s1 = refs
        core = lax.axis_index("c")
        sub = lax.axis_index("s")
        wid = core * NS + sub
        r0 = sub * rpt

        pltpu.sync_copy(zer.at[pl.ds(r0, rpt)], acc.at[pl.ds(r0, rpt)])
        if with_deg:
            @pl.when(sub == 0)
            def _zero_deg():
                pltpu.sync_copy(zer1, accd)
            for k in range(CHUNK // 16):
                ones[pl.ds(16 * k, 16)] = jnp.ones((16,), jnp.float32)
        plsc.subcore_barrier()

        sems = (s0, s1)

        def load_group(j):
            q = j // GRP
            s = q % 2
            pltpu.sync_copy(colh.at[pl.ds(wid * nch + q * GRP, GRP)], colv.at[s])
            pltpu.sync_copy(rowh.at[pl.ds(wid * nch + q * GRP, GRP)], rowv.at[s])

        def start_gather(j, b):
            pltpu.async_copy(
                tab.at[colv.at[(j // GRP) % 2, j % GRP]], rowsv.at[b], sems[b])

        def wait_gather(b):
            pltpu.make_async_copy(tab.at[colv.at[0, 0]], rowsv.at[b],
                                  sems[b]).wait()

        def scatter_add(j, b):
            ridx = rowv.at[(j // GRP) % 2, j % GRP]
            pltpu.sync_copy(rowsv.at[b], acc.at[ridx], add=True)
            if with_deg:
                pltpu.sync_copy(ones, accd.at[ridx], add=True)

        load_group(0)
        start_gather(0, 0)

        def body(i, _):
            g = 2 * i
            start_gather(g + 1, 1)
            wait_gather(0)
            scatter_add(g, 0)

            @pl.when(jnp.logical_and(g % GRP == 2, g + 2 < nch))
            def _load_next():
                load_group(g + 2)

            @pl.when(g + 2 < nch)
            def _prefetch():
                start_gather(g + 2, 0)

            wait_gather(1)
            scatter_add(g + 1, 1)
            return ()

        lax.fori_loop(0, nch // 2, body, ())

        plsc.subcore_barrier()
        pltpu.sync_copy(acc.at[pl.ds(r0, rpt)], out.at[core].at[pl.ds(r0, rpt)])
        if with_deg:
            @pl.when(sub == 0)
            def _pub_deg():
                pltpu.sync_copy(accd, outd.at[core])

    return sc_kernel


def _layer1_tc(p, cnt2d, w1t, b1):
    grid = N_PAD // D

    def body(p_ref, c_ref, w_ref, b_ref, h_ref, rd_ref):
        s = p_ref[0] + p_ref[1]
        c = c_ref[0, 0] + c_ref[1, 0]
        rd_row = 1.0 / jnp.maximum(c, 1.0)
        ident = (lax.broadcasted_iota(jnp.int32, (D, D), 0)
                 == lax.broadcasted_iota(jnp.int32, (D, D), 1)
                 ).astype(jnp.float32)
        rd_col = lax.dot_general(ident, rd_row, (((1,), (1,)), ((), ())),
                                 preferred_element_type=jnp.float32)
        h = jnp.dot(s * rd_col, w_ref[...], preferred_element_type=jnp.float32)
        h_ref[...] = jnp.maximum(h + b_ref[...], 0.0)
        rd_ref[...] = jnp.broadcast_to(rd_col, (D, D))

    return pl.pallas_call(
        body,
        grid=(grid,),
        in_specs=[
            pl.BlockSpec((NC, D, D), lambda i: (0, i, 0)),
            pl.BlockSpec((NC, 1, 1, D), lambda i: (0, i, 0, 0)),
            pl.BlockSpec((D, D), lambda i: (0, 0)),
            pl.BlockSpec((1, D), lambda i: (0, 0)),
        ],
        out_specs=[
            pl.BlockSpec((D, D), lambda i: (i, 0)),
            pl.BlockSpec((D, D), lambda i: (i, 0)),
        ],
        out_shape=[
            jax.ShapeDtypeStruct((N_PAD, D), jnp.float32),
            jax.ShapeDtypeStruct((N_PAD, D), jnp.float32),
        ],
    )(p, cnt2d, w1t, b1)


def _layer2_tc(p, rdeg, w2t, b2):
    grid = N_PAD // BS2

    def body(p_ref, rd_ref, w_ref, b_ref, o_ref):
        s = (p_ref[0] + p_ref[1]) * rd_ref[...]
        o = jnp.dot(s, w_ref[...], preferred_element_type=jnp.float32)
        o_ref[...] = jnp.maximum(o + b_ref[...], 0.0)

    return pl.pallas_call(
        body,
        grid=(grid,),
        in_specs=[
            pl.BlockSpec((NC, BS2, D), lambda i: (0, i, 0)),
            pl.BlockSpec((BS2, D), lambda i: (i, 0)),
            pl.BlockSpec((D, D), lambda i: (0, 0)),
            pl.BlockSpec((1, D), lambda i: (0, 0)),
        ],
        out_specs=pl.BlockSpec((BS2, D), lambda i: (i, 0)),
        out_shape=jax.ShapeDtypeStruct((N_PAD, D), jnp.float32),
    )(p, rdeg, w2t, b2)


@jax.jit
def kernel(x, edge_index, W1, b1, W2, b2):
    n = x.shape[0]
    e = edge_index.shape[1]

    row = edge_index[0].astype(jnp.int32)
    col = edge_index[1].astype(jnp.int32)

    e_pad = -(-e // (NWK * 2 * CHUNK)) * (NWK * 2 * CHUNK)
    npad = e_pad - e
    pad_rows = n + (jnp.arange(npad, dtype=jnp.int32) % (N_PAD - n))
    pad_cols = jnp.arange(npad, dtype=jnp.int32) % n
    row_p = jnp.concatenate([row, pad_rows]).reshape(e_pad // CHUNK, CHUNK)
    col_p = jnp.concatenate([col, pad_cols]).reshape(e_pad // CHUNK, CHUNK)

    z_d = jnp.zeros((N_PAD, D), dtype=jnp.float32)
    z_1 = jnp.zeros((N_PAD,), dtype=jnp.float32)

    sc1 = _make_sc_scatter(e_pad, with_deg=True)
    sc2 = _make_sc_scatter(e_pad, with_deg=False)

    p1, cnt = sc1(x, col_p, row_p, z_d, z_1)
    cnt2d = cnt.reshape(NC, N_PAD // D, 1, D)
    h, rdeg = _layer1_tc(p1, cnt2d, W1.T, b1.reshape(1, D))

    (p2,) = sc2(h, col_p, row_p, z_d, z_1)
    out = _layer2_tc(p2, rdeg, W2.T, b2.reshape(1, D))

    return out[:n]


# speedup vs baseline: 9.5390x; 1.0842x over previous
import functools

import jax
import jax.numpy as jnp
from jax import lax
from jax.experimental import pallas as pl
from jax.experimental.pallas import tpu as pltpu
from jax.experimental.pallas import tpu_sc as plsc

N_NODES = 10000
D = 128

NC = 2
NS = 16
NWK = NC * NS

CHUNK = 128
GRP = 4
N_PAD = 10112
BS2 = 632


def _make_sc_scatter(e_pad: int, with_deg: bool):
    npw = e_pad // NWK
    nch = npw // CHUNK
    assert npw % CHUNK == 0 and nch % GRP == 0 and nch % 2 == 0
    rpt = N_PAD // NS

    mesh = plsc.VectorSubcoreMesh(
        core_axis_name="c", subcore_axis_name="s", num_cores=NC, num_subcores=NS)

    out_type = [jax.ShapeDtypeStruct((NC, N_PAD, D), jnp.float32)]
    scratch = [
        pltpu.VMEM((2, GRP, CHUNK), jnp.int32),
        pltpu.VMEM((2, GRP, CHUNK), jnp.int32),
        pltpu.VMEM((2, CHUNK, D), jnp.float32),
        pltpu.VMEM_SHARED((N_PAD, D), jnp.float32),
        pltpu.SemaphoreType.DMA,
        pltpu.SemaphoreType.DMA,
    ]
    if with_deg:
        out_type.append(jax.ShapeDtypeStruct((NC, N_PAD), jnp.float32))
        scratch += [
            pltpu.VMEM_SHARED((N_PAD,), jnp.float32),
            pltpu.VMEM((CHUNK,), jnp.float32),
        ]

    @functools.partial(
        pl.kernel,
        out_type=out_type,
        mesh=mesh,
        scratch_types=scratch,
        compiler_params=pltpu.CompilerParams(use_tc_tiling_on_sc=False),
    )
    def sc_kernel(tab, colh, rowh, zer, zer1, *refs):
        if with_deg:
            out, outd, colv, rowv, rowsv, acc, s0, s1, accd, ones = refs
        else:
            out, colv, rowv, rowsv, acc, s0, s1 = refs
        core = lax.axis_index("c")
        sub = lax.axis_index("s")
        wid = core * NS + sub
        r0 = sub * rpt

        pltpu.sync_copy(zer.at[pl.ds(r0, rpt)], acc.at[pl.ds(r0, rpt)])
        if with_deg:
            @pl.when(sub == 0)
            def _zero_deg():
                pltpu.sync_copy(zer1, accd)
            for k in range(CHUNK // 16):
                ones[pl.ds(16 * k, 16)] = jnp.ones((16,), jnp.float32)
        plsc.subcore_barrier()

        sems = (s0, s1)

        def load_group(j):
            q = j // GRP
            s = q % 2
            pltpu.sync_copy(colh.at[pl.ds(wid * nch + q * GRP, GRP)], colv.at[s])
            pltpu.sync_copy(rowh.at[pl.ds(wid * nch + q * GRP, GRP)], rowv.at[s])

        def start_gather(j, b):
            pltpu.async_copy(
                tab.at[colv.at[(j // GRP) % 2, j % GRP]], rowsv.at[b], sems[b])

        def wait_gather(b):
            pltpu.make_async_copy(tab.at[colv.at[0, 0]], rowsv.at[b],
                                  sems[b]).wait()

        def scatter_add(j, b):
            ridx = rowv.at[(j // GRP) % 2, j % GRP]
            pltpu.sync_copy(rowsv.at[b], acc.at[ridx], add=True)
            if with_deg:
                pltpu.sync_copy(ones, accd.at[ridx], add=True)

        load_group(0)
        start_gather(0, 0)

        def body(i, _):
            g = 2 * i
            start_gather(g + 1, 1)
            wait_gather(0)
            scatter_add(g, 0)

            @pl.when(jnp.logical_and(g % GRP == 2, g + 2 < nch))
            def _load_next():
                load_group(g + 2)

            @pl.when(g + 2 < nch)
            def _prefetch():
                start_gather(g + 2, 0)

            wait_gather(1)
            scatter_add(g + 1, 1)
            return ()

        lax.fori_loop(0, nch // 2, body, ())

        plsc.subcore_barrier()
        pltpu.sync_copy(acc.at[pl.ds(r0, rpt)], out.at[core].at[pl.ds(r0, rpt)])
        if with_deg:
            @pl.when(sub == 0)
            def _pub_deg():
                pltpu.sync_copy(accd, outd.at[core])

    return sc_kernel


def _rdeg_tc(cnt2d):
    nr = N_PAD // D

    def body(c_ref, rd_ref):
        rd_ref[...] = 1.0 / jnp.maximum(c_ref[0] + c_ref[1], 1.0)

    return pl.pallas_call(
        body,
        out_shape=jax.ShapeDtypeStruct((nr, D), jnp.float32),
    )(cnt2d)


def _mlp_tc(p, rdcol, wt, b):
    grid = N_PAD // BS2

    def body(p_ref, rd_ref, w_ref, b_ref, o_ref):
        s = (p_ref[0] + p_ref[1]) * rd_ref[...]
        o = jnp.dot(s, w_ref[...], preferred_element_type=jnp.float32)
        o_ref[...] = jnp.maximum(o + b_ref[...], 0.0)

    return pl.pallas_call(
        body,
        grid=(grid,),
        in_specs=[
            pl.BlockSpec((NC, BS2, D), lambda i: (0, i, 0)),
            pl.BlockSpec((BS2, 1), lambda i: (i, 0)),
            pl.BlockSpec((D, D), lambda i: (0, 0)),
            pl.BlockSpec((1, D), lambda i: (0, 0)),
        ],
        out_specs=pl.BlockSpec((BS2, D), lambda i: (i, 0)),
        out_shape=jax.ShapeDtypeStruct((N_PAD, D), jnp.float32),
    )(p, rdcol, wt, b)


@jax.jit
def kernel(x, edge_index, W1, b1, W2, b2):
    n = x.shape[0]
    e = edge_index.shape[1]

    row = edge_index[0].astype(jnp.int32)
    col = edge_index[1].astype(jnp.int32)

    e_pad = -(-e // (NWK * 2 * CHUNK)) * (NWK * 2 * CHUNK)
    npad = e_pad - e
    pad_rows = n + (jnp.arange(npad, dtype=jnp.int32) % (N_PAD - n))
    pad_cols = jnp.arange(npad, dtype=jnp.int32) % n
    row_p = jnp.concatenate([row, pad_rows]).reshape(e_pad // CHUNK, CHUNK)
    col_p = jnp.concatenate([col, pad_cols]).reshape(e_pad // CHUNK, CHUNK)

    z_d = jnp.zeros((N_PAD, D), dtype=jnp.float32)
    z_1 = jnp.zeros((N_PAD,), dtype=jnp.float32)

    sc1 = _make_sc_scatter(e_pad, with_deg=True)
    sc2 = _make_sc_scatter(e_pad, with_deg=False)

    p1, cnt = sc1(x, col_p, row_p, z_d, z_1)
    rd2 = _rdeg_tc(cnt.reshape(NC, N_PAD // D, D))
    rdcol = rd2.reshape(N_PAD, 1)
    h = _mlp_tc(p1, rdcol, W1.T, b1.reshape(1, D))

    (p2,) = sc2(h, col_p, row_p, z_d, z_1)
    out = _mlp_tc(p2, rdcol, W2.T, b2.reshape(1, D))

    return out[:n]
